# baseline (speedup 1.0000x reference)
import functools

import jax
import jax.numpy as jnp
from jax import lax
from jax.experimental import pallas as pl
from jax.experimental.pallas import tpu as pltpu
from jax.experimental.pallas import tpu_sc as plsc

N_NODES = 10000
N_EDGES = 320000
DIM = 128
N_GRAPHS = 256

NC = 2
NS = 16
NW = NC * NS

NPAD = 10240
EPAD = 327680
EB = EPAD // 128
EB64 = EPAD // 64
CW = EB // NW
CW64 = EB64 // NW
TROWS = NPAD // NS

BR = 512
NB = NPAD // BR
CB = BR // 128

_mesh = plsc.VectorSubcoreMesh(core_axis_name="c", subcore_axis_name="s",
                               num_cores=NC, num_subcores=NS)


CW2 = EB // NS


@functools.partial(
    pl.kernel,
    out_type=jax.ShapeDtypeStruct((2, NPAD, 128), jnp.float32),
    mesh=_mesh,
    scratch_types=[
        pltpu.VMEM((CW2, 128), jnp.int32),
        pltpu.VMEM((128, 128), jnp.float32),
        pltpu.VMEM_SHARED((NPAD, 128), jnp.float32),
    ],
)
def _sc_degrees(idx_hbm, ones_hbm, zeros_hbm, out_hbm,
                idx_v, ones_v, deg_sh):
    c = lax.axis_index("c")
    s = lax.axis_index("s")
    pltpu.sync_copy(zeros_hbm.at[pl.ds(s * TROWS, TROWS)],
                    deg_sh.at[pl.ds(s * TROWS, TROWS)])
    pltpu.sync_copy(ones_hbm, ones_v)
    pltpu.sync_copy(idx_hbm.at[c, pl.ds(s * CW2, CW2)], idx_v)
    plsc.subcore_barrier()

    @pl.loop(0, CW2)
    def _chunk(j):
        pltpu.sync_copy(ones_v, deg_sh.at[idx_v.at[j]], add=True)

    plsc.subcore_barrier()
    pltpu.sync_copy(deg_sh.at[pl.ds(s * TROWS, TROWS)],
                    out_hbm.at[c, pl.ds(s * TROWS, TROWS)])


R0 = 120
R1 = 160 - R0
RMAX = max(R0, R1)


@functools.partial(
    pl.kernel,
    out_type=jax.ShapeDtypeStruct((NC, NPAD, DIM), jnp.float32),
    mesh=_mesh,
    scratch_types=[
        pltpu.VMEM((RMAX, 128), jnp.int32),
        pltpu.VMEM((RMAX, 128), jnp.int32),
        pltpu.VMEM((64, DIM), jnp.float32),
        pltpu.VMEM((64, DIM), jnp.float32),
        pltpu.VMEM_SHARED((NPAD, DIM), jnp.float32),
        pltpu.SemaphoreType.DMA,
        pltpu.SemaphoreType.DMA,
        pltpu.SemaphoreType.DMA,
        pltpu.SemaphoreType.DMA,
    ],
)
def _sc_scatter(m_hbm, src_hbm, dst_hbm, zeros_hbm, out_hbm,
                src_v, dst_v, r0, r1, agg_sh, g0, g1, s0, s1):
    rows = (r0, r1)
    gsem = (g0, g1)
    ssem = (s0, s1)
    c = lax.axis_index("c")
    s = lax.axis_index("s")
    pltpu.sync_copy(zeros_hbm.at[pl.ds(s * TROWS, TROWS)],
                    agg_sh.at[pl.ds(s * TROWS, TROWS)])

    def _run(base_rows, nrows):
        n64 = 2 * nrows
        pltpu.sync_copy(src_hbm.at[pl.ds(base_rows, nrows)],
                        src_v.at[pl.ds(0, nrows)])
        pltpu.sync_copy(dst_hbm.at[pl.ds(base_rows, nrows)],
                        dst_v.at[pl.ds(0, nrows)])

        def _sidx(k, b):
            return src_v.at[k // 2, pl.ds(b * 64, 64)]

        def _didx(k, b):
            return dst_v.at[k // 2, pl.ds(b * 64, 64)]

        pltpu.async_copy(m_hbm.at[_sidx(0, 0)], rows[0], gsem[0])

        @pl.loop(0, n64, step=2)
        def _blk(j):
            for b in range(2):
                k = j + b
                o = b ^ 1

                @pl.when(k >= 1)
                def _free_other():
                    pltpu.make_async_copy(
                        rows[o], agg_sh.at[_didx(k - 1, o)], ssem[o]).wait()

                @pl.when(k + 1 < n64)
                def _gather_next():
                    pltpu.async_copy(m_hbm.at[_sidx(k + 1, o)], rows[o],
                                     gsem[o])

                pltpu.make_async_copy(m_hbm.at[_sidx(k, b)], rows[b],
                                      gsem[b]).wait()
                pltpu.async_copy(rows[b], agg_sh.at[_didx(k, b)], ssem[b],
                                 add=True)

        pltpu.make_async_copy(rows[1], agg_sh.at[_didx(n64 - 1, 1)],
                              ssem[1]).wait()

    @pl.when(c == 0)
    def _core0():
        _run(s * R0, R0)

    @pl.when(c == 1)
    def _core1():
        _run(16 * R0 + s * R1, R1)

    plsc.subcore_barrier()
    pltpu.sync_copy(agg_sh.at[pl.ds(s * TROWS, TROWS)],
                    out_hbm.at[c, pl.ds(s * TROWS, TROWS)])


def _embed_a_body(x_ref, wi_ref, w1_ref, h0_ref, u1_ref):
    h0 = jnp.dot(x_ref[...], wi_ref[...], preferred_element_type=jnp.float32)
    h0_ref[...] = h0
    u1_ref[...] = jnp.dot(h0, w1_ref[...], preferred_element_type=jnp.float32)


def _tc_embed_a(x_pad, w_init, w1):
    return pl.pallas_call(
        _embed_a_body,
        grid=(NB,),
        in_specs=[
            pl.BlockSpec((BR, DIM), lambda j: (j, 0)),
            pl.BlockSpec((DIM, DIM), lambda j: (0, 0)),
            pl.BlockSpec((DIM, DIM), lambda j: (0, 0)),
        ],
        out_specs=[
            pl.BlockSpec((BR, DIM), lambda j: (j, 0)),
            pl.BlockSpec((BR, DIM), lambda j: (j, 0)),
        ],
        out_shape=[
            jax.ShapeDtypeStruct((NPAD, DIM), jnp.float32),
            jax.ShapeDtypeStruct((NPAD, DIM), jnp.float32),
        ],
    )(x_pad, w_init, w1)


def _embed_b_body(degp_ref, u1_ref, m1_ref, on_ref, in_ref):
    do = jnp.maximum(degp_ref[0, :, 0:16], 1.0)
    di = jnp.maximum(degp_ref[1, :, 0:16], 1.0)
    onb = lax.rsqrt(do)
    on_ref[...] = onb
    in_ref[...] = lax.rsqrt(di)
    m1_ref[...] = u1_ref[...] * onb[:, 0:1]


def _tc_embed_b(degp, u1):
    return pl.pallas_call(
        _embed_b_body,
        grid=(NB,),
        in_specs=[
            pl.BlockSpec((2, BR, 128), lambda j: (0, j, 0)),
            pl.BlockSpec((BR, DIM), lambda j: (j, 0)),
        ],
        out_specs=[
            pl.BlockSpec((BR, DIM), lambda j: (j, 0)),
            pl.BlockSpec((BR, 16), lambda j: (j, 0)),
            pl.BlockSpec((BR, 16), lambda j: (j, 0)),
        ],
        out_shape=[
            jax.ShapeDtypeStruct((NPAD, DIM), jnp.float32),
            jax.ShapeDtypeStruct((NPAD, 16), jnp.float32),
            jax.ShapeDtypeStruct((NPAD, 16), jnp.float32),
        ],
    )(degp, u1)


def _layer_body(p_ref, in_ref, b_ref, h_ref, wr_ref, br_ref, on_ref, w_ref,
                hout_ref, mout_ref):
    agg = p_ref[0] + p_ref[1]
    new = jnp.maximum(agg * in_ref[:, 0:1] + b_ref[...], 0.0)
    res = jnp.maximum(
        jnp.dot(h_ref[...], wr_ref[...], preferred_element_type=jnp.float32)
        + br_ref[...], 0.0)
    h = new + res
    hout_ref[...] = h
    mout_ref[...] = jnp.dot(h * on_ref[:, 0:1], w_ref[...],
                            preferred_element_type=jnp.float32)


def _tc_layer(aggp, inorm, b, h_prev, wr, br, onorm, w_next):
    return pl.pallas_call(
        _layer_body,
        grid=(NB,),
        in_specs=[
            pl.BlockSpec((NC, BR, DIM), lambda j: (0, j, 0)),
            pl.BlockSpec((BR, 16), lambda j: (j, 0)),
            pl.BlockSpec((1, DIM), lambda j: (0, 0)),
            pl.BlockSpec((BR, DIM), lambda j: (j, 0)),
            pl.BlockSpec((DIM, DIM), lambda j: (0, 0)),
            pl.BlockSpec((1, DIM), lambda j: (0, 0)),
            pl.BlockSpec((BR, 16), lambda j: (j, 0)),
            pl.BlockSpec((DIM, DIM), lambda j: (0, 0)),
        ],
        out_specs=[
            pl.BlockSpec((BR, DIM), lambda j: (j, 0)),
            pl.BlockSpec((BR, DIM), lambda j: (j, 0)),
        ],
        out_shape=[
            jax.ShapeDtypeStruct((NPAD, DIM), jnp.float32),
            jax.ShapeDtypeStruct((NPAD, DIM), jnp.float32),
        ],
    )(aggp, inorm, b, h_prev, wr, br, onorm, w_next)


def _final_body(p_ref, in_ref, b_ref, h_ref, wr_ref, br_ref, gid_ref,
                out_ref, acc_ref, cnt_ref):
    j = pl.program_id(0)

    @pl.when(j == 0)
    def _init():
        acc_ref[...] = jnp.zeros_like(acc_ref)
        cnt_ref[...] = jnp.zeros_like(cnt_ref)

    agg = p_ref[0] + p_ref[1]
    new = jnp.maximum(agg * in_ref[:, 0:1] + b_ref[...], 0.0)
    res = jnp.maximum(
        jnp.dot(h_ref[...], wr_ref[...], preferred_element_type=jnp.float32)
        + br_ref[...], 0.0)
    h = new + res
    gids = gid_ref[:, 0:1]
    oh = (gids == lax.broadcasted_iota(jnp.int32, (BR, N_GRAPHS), 1)
          ).astype(jnp.float32)
    acc_ref[...] += lax.dot_general(
        oh, h, (((0,), (0,)), ((), ())),
        preferred_element_type=jnp.float32)
    cnt_ref[...] += lax.dot_general(
        oh, jnp.ones((BR, DIM), jnp.float32), (((0,), (0,)), ((), ())),
        preferred_element_type=jnp.float32)

    @pl.when(j == NB - 1)
    def _fin():
        out_ref[...] = acc_ref[...] / jnp.maximum(cnt_ref[...], 1.0)


def _tc_final(aggp, inorm, b, h_prev, wr, br, gid2):
    return pl.pallas_call(
        _final_body,
        grid=(NB,),
        in_specs=[
            pl.BlockSpec((NC, BR, DIM), lambda j: (0, j, 0)),
            pl.BlockSpec((BR, 16), lambda j: (j, 0)),
            pl.BlockSpec((1, DIM), lambda j: (0, 0)),
            pl.BlockSpec((BR, DIM), lambda j: (j, 0)),
            pl.BlockSpec((DIM, DIM), lambda j: (0, 0)),
            pl.BlockSpec((1, DIM), lambda j: (0, 0)),
            pl.BlockSpec((BR, 16), lambda j: (j, 0)),
        ],
        out_specs=pl.BlockSpec((N_GRAPHS, DIM), lambda j: (0, 0)),
        out_shape=jax.ShapeDtypeStruct((N_GRAPHS, DIM), jnp.float32),
        scratch_shapes=[
            pltpu.VMEM((N_GRAPHS, DIM), jnp.float32),
            pltpu.VMEM((N_GRAPHS, DIM), jnp.float32),
        ],
        compiler_params=pltpu.CompilerParams(
            dimension_semantics=("arbitrary",)),
    )(aggp, inorm, b, h_prev, wr, br, gid2)


def kernel(x, edge_index, graph_ids, W_init,
           W1, b1, Wr1, br1, W2, b2, Wr2, br2, W3, b3, Wr3, br3):
    src = edge_index[0]
    dst = edge_index[1]
    pad_e = jnp.full((EPAD - N_EDGES,), NPAD - 1, jnp.int32)
    src_pad = jnp.concatenate([src, pad_e])
    dst_pad = jnp.concatenate([dst, pad_e])
    src2d = src_pad.reshape(EB, 128)
    dst2d = dst_pad.reshape(EB, 128)
    src64 = src_pad.reshape(EB64, 64)
    dst64 = dst_pad.reshape(EB64, 64)
    x_pad = jnp.pad(x, ((0, NPAD - N_NODES), (0, 0)))
    gid_pad = jnp.concatenate(
        [graph_ids, jnp.full((NPAD - N_NODES,), -1, jnp.int32)])
    gid2 = jnp.broadcast_to(gid_pad[:, None], (NPAD, 16))
    ones128 = jnp.ones((128, 128), jnp.float32)
    zeros_nd = jnp.zeros((NPAD, DIM), jnp.float32)
    b1r = b1.reshape(1, DIM)
    br1r = br1.reshape(1, DIM)
    b2r = b2.reshape(1, DIM)
    br2r = br2.reshape(1, DIM)
    b3r = b3.reshape(1, DIM)
    br3r = br3.reshape(1, DIM)

    idx_all = jnp.stack([src2d, dst2d])
    h0, u1 = _tc_embed_a(x_pad, W_init, W1)
    degp = _sc_degrees(idx_all, ones128, zeros_nd)
    m1, onorm, inorm = _tc_embed_b(degp, u1)
    aggp = _sc_scatter(m1, src2d, dst2d, zeros_nd)
    h1, m2 = _tc_layer(aggp, inorm, b1r, h0, Wr1, br1r, onorm, W2)
    aggp = _sc_scatter(m2, src2d, dst2d, zeros_nd)
    h2, m3 = _tc_layer(aggp, inorm, b2r, h1, Wr2, br2r, onorm, W3)
    aggp = _sc_scatter(m3, src2d, dst2d, zeros_nd)
    return _tc_final(aggp, inorm, b3r, h2, Wr3, br3r, gid2)


# speedup vs baseline: 1.1168x; 1.1168x over previous
import functools

import jax
import jax.numpy as jnp
from jax import lax
from jax.experimental import pallas as pl
from jax.experimental.pallas import tpu as pltpu
from jax.experimental.pallas import tpu_sc as plsc

N_NODES = 10000
N_EDGES = 320000
DIM = 128
N_GRAPHS = 256

NC = 2
NS = 16
NW = NC * NS

NPAD = 10240
EPAD = 327680
EB = EPAD // 128
EB64 = EPAD // 64
CW = EB // NW
CW64 = EB64 // NW
TROWS = NPAD // NS

BR = 512
NB = NPAD // BR
CB = BR // 128

_mesh = plsc.VectorSubcoreMesh(core_axis_name="c", subcore_axis_name="s",
                               num_cores=NC, num_subcores=NS)


CW2 = EB // NS


@functools.partial(
    pl.kernel,
    out_type=jax.ShapeDtypeStruct((2, NPAD, 128), jnp.float32),
    mesh=_mesh,
    scratch_types=[
        pltpu.VMEM((CW2, 128), jnp.int32),
        pltpu.VMEM((128, 128), jnp.float32),
        pltpu.VMEM_SHARED((NPAD, 128), jnp.float32),
        pltpu.SemaphoreType.DMA,
    ],
)
def _sc_degrees(idx_hbm, ones_hbm, zeros_hbm, out_hbm,
                idx_v, ones_v, deg_sh, dsem):
    c = lax.axis_index("c")
    s = lax.axis_index("s")
    pltpu.sync_copy(zeros_hbm.at[pl.ds(s * TROWS, TROWS)],
                    deg_sh.at[pl.ds(s * TROWS, TROWS)])
    pltpu.sync_copy(ones_hbm, ones_v)
    pltpu.sync_copy(idx_hbm.at[c, pl.ds(s * CW2, CW2)], idx_v)
    plsc.subcore_barrier()

    @pl.loop(0, CW2, step=8)
    def _wave(j):
        for i in range(8):
            pltpu.async_copy(ones_v, deg_sh.at[idx_v.at[j + i]], dsem,
                             add=True)
        for i in range(8):
            pltpu.make_async_copy(ones_v, deg_sh.at[idx_v.at[j + i]],
                                  dsem).wait()

    plsc.subcore_barrier()
    pltpu.sync_copy(deg_sh.at[pl.ds(s * TROWS, TROWS)],
                    out_hbm.at[c, pl.ds(s * TROWS, TROWS)])


R0 = 120
R1 = 160 - R0
RMAX = max(R0, R1)


@functools.partial(
    pl.kernel,
    out_type=jax.ShapeDtypeStruct((NC, NPAD, DIM), jnp.float32),
    mesh=_mesh,
    scratch_types=[
        pltpu.VMEM((RMAX, 128), jnp.int32),
        pltpu.VMEM((RMAX, 128), jnp.int32),
        pltpu.VMEM((64, DIM), jnp.float32),
        pltpu.VMEM((64, DIM), jnp.float32),
        pltpu.VMEM_SHARED((NPAD, DIM), jnp.float32),
        pltpu.SemaphoreType.DMA,
        pltpu.SemaphoreType.DMA,
        pltpu.SemaphoreType.DMA,
        pltpu.SemaphoreType.DMA,
    ],
)
def _sc_scatter(m_hbm, src_hbm, dst_hbm, zeros_hbm, out_hbm,
                src_v, dst_v, r0, r1, agg_sh, g0, g1, s0, s1):
    rows = (r0, r1)
    gsem = (g0, g1)
    ssem = (s0, s1)
    c = lax.axis_index("c")
    s = lax.axis_index("s")
    pltpu.sync_copy(zeros_hbm.at[pl.ds(s * TROWS, TROWS)],
                    agg_sh.at[pl.ds(s * TROWS, TROWS)])

    def _run(base_rows, nrows):
        n64 = 2 * nrows
        pltpu.sync_copy(src_hbm.at[pl.ds(base_rows, nrows)],
                        src_v.at[pl.ds(0, nrows)])
        pltpu.sync_copy(dst_hbm.at[pl.ds(base_rows, nrows)],
                        dst_v.at[pl.ds(0, nrows)])

        def _sidx(k, b):
            return src_v.at[k // 2, pl.ds(b * 64, 64)]

        def _didx(k, b):
            return dst_v.at[k // 2, pl.ds(b * 64, 64)]

        pltpu.async_copy(m_hbm.at[_sidx(0, 0)], rows[0], gsem[0])

        @pl.loop(0, n64, step=2)
        def _blk(j):
            for b in range(2):
                k = j + b
                o = b ^ 1

                @pl.when(k >= 1)
                def _free_other():
                    pltpu.make_async_copy(
                        rows[o], agg_sh.at[_didx(k - 1, o)], ssem[o]).wait()

                @pl.when(k + 1 < n64)
                def _gather_next():
                    pltpu.async_copy(m_hbm.at[_sidx(k + 1, o)], rows[o],
                                     gsem[o])

                pltpu.make_async_copy(m_hbm.at[_sidx(k, b)], rows[b],
                                      gsem[b]).wait()
                pltpu.async_copy(rows[b], agg_sh.at[_didx(k, b)], ssem[b],
                                 add=True)

        pltpu.make_async_copy(rows[1], agg_sh.at[_didx(n64 - 1, 1)],
                              ssem[1]).wait()

    @pl.when(c == 0)
    def _core0():
        _run(s * R0, R0)

    @pl.when(c == 1)
    def _core1():
        _run(16 * R0 + s * R1, R1)

    plsc.subcore_barrier()
    pltpu.sync_copy(agg_sh.at[pl.ds(s * TROWS, TROWS)],
                    out_hbm.at[c, pl.ds(s * TROWS, TROWS)])


def _embed_body(degp_ref, x_ref, wi_ref, w1_ref,
                h0_ref, m1_ref, on_ref, in_ref):
    do = jnp.maximum(degp_ref[0, :, 0:16], 1.0)
    di = jnp.maximum(degp_ref[1, :, 0:16], 1.0)
    onb = lax.rsqrt(do)
    on_ref[...] = onb
    in_ref[...] = lax.rsqrt(di)
    h0 = jnp.dot(x_ref[...], wi_ref[...], preferred_element_type=jnp.float32)
    h0_ref[...] = h0
    m1_ref[...] = jnp.dot(h0 * onb[:, 0:1], w1_ref[...],
                          preferred_element_type=jnp.float32)


def _tc_embed(degp, x_pad, w_init, w1):
    return pl.pallas_call(
        _embed_body,
        grid=(NB,),
        in_specs=[
            pl.BlockSpec((2, BR, 128), lambda j: (0, j, 0)),
            pl.BlockSpec((BR, DIM), lambda j: (j, 0)),
            pl.BlockSpec((DIM, DIM), lambda j: (0, 0)),
            pl.BlockSpec((DIM, DIM), lambda j: (0, 0)),
        ],
        out_specs=[
            pl.BlockSpec((BR, DIM), lambda j: (j, 0)),
            pl.BlockSpec((BR, DIM), lambda j: (j, 0)),
            pl.BlockSpec((BR, 16), lambda j: (j, 0)),
            pl.BlockSpec((BR, 16), lambda j: (j, 0)),
        ],
        out_shape=[
            jax.ShapeDtypeStruct((NPAD, DIM), jnp.float32),
            jax.ShapeDtypeStruct((NPAD, DIM), jnp.float32),
            jax.ShapeDtypeStruct((NPAD, 16), jnp.float32),
            jax.ShapeDtypeStruct((NPAD, 16), jnp.float32),
        ],
    )(degp, x_pad, w_init, w1)


def _layer_body(p_ref, in_ref, b_ref, h_ref, wr_ref, br_ref, on_ref, w_ref,
                hout_ref, mout_ref):
    agg = p_ref[0] + p_ref[1]
    new = jnp.maximum(agg * in_ref[:, 0:1] + b_ref[...], 0.0)
    res = jnp.maximum(
        jnp.dot(h_ref[...], wr_ref[...], preferred_element_type=jnp.float32)
        + br_ref[...], 0.0)
    h = new + res
    hout_ref[...] = h
    mout_ref[...] = jnp.dot(h * on_ref[:, 0:1], w_ref[...],
                            preferred_element_type=jnp.float32)


def _tc_layer(aggp, inorm, b, h_prev, wr, br, onorm, w_next):
    return pl.pallas_call(
        _layer_body,
        grid=(NB,),
        in_specs=[
            pl.BlockSpec((NC, BR, DIM), lambda j: (0, j, 0)),
            pl.BlockSpec((BR, 16), lambda j: (j, 0)),
            pl.BlockSpec((1, DIM), lambda j: (0, 0)),
            pl.BlockSpec((BR, DIM), lambda j: (j, 0)),
            pl.BlockSpec((DIM, DIM), lambda j: (0, 0)),
            pl.BlockSpec((1, DIM), lambda j: (0, 0)),
            pl.BlockSpec((BR, 16), lambda j: (j, 0)),
            pl.BlockSpec((DIM, DIM), lambda j: (0, 0)),
        ],
        out_specs=[
            pl.BlockSpec((BR, DIM), lambda j: (j, 0)),
            pl.BlockSpec((BR, DIM), lambda j: (j, 0)),
        ],
        out_shape=[
            jax.ShapeDtypeStruct((NPAD, DIM), jnp.float32),
            jax.ShapeDtypeStruct((NPAD, DIM), jnp.float32),
        ],
    )(aggp, inorm, b, h_prev, wr, br, onorm, w_next)


def _final_body(p_ref, in_ref, b_ref, h_ref, wr_ref, br_ref, gid_ref,
                out_ref, acc_ref, cnt_ref):
    j = pl.program_id(0)

    @pl.when(j == 0)
    def _init():
        acc_ref[...] = jnp.zeros_like(acc_ref)
        cnt_ref[...] = jnp.zeros_like(cnt_ref)

    agg = p_ref[0] + p_ref[1]
    new = jnp.maximum(agg * in_ref[:, 0:1] + b_ref[...], 0.0)
    res = jnp.maximum(
        jnp.dot(h_ref[...], wr_ref[...], preferred_element_type=jnp.float32)
        + br_ref[...], 0.0)
    h = new + res
    gids = gid_ref[:, 0:1]
    oh = (gids == lax.broadcasted_iota(jnp.int32, (BR, N_GRAPHS), 1)
          ).astype(jnp.float32)
    acc_ref[...] += lax.dot_general(
        oh, h, (((0,), (0,)), ((), ())),
        preferred_element_type=jnp.float32)
    cnt_ref[...] += lax.dot_general(
        oh, jnp.ones((BR, DIM), jnp.float32), (((0,), (0,)), ((), ())),
        preferred_element_type=jnp.float32)

    @pl.when(j == NB - 1)
    def _fin():
        out_ref[...] = acc_ref[...] / jnp.maximum(cnt_ref[...], 1.0)


def _tc_final(aggp, inorm, b, h_prev, wr, br, gid2):
    return pl.pallas_call(
        _final_body,
        grid=(NB,),
        in_specs=[
            pl.BlockSpec((NC, BR, DIM), lambda j: (0, j, 0)),
            pl.BlockSpec((BR, 16), lambda j: (j, 0)),
            pl.BlockSpec((1, DIM), lambda j: (0, 0)),
            pl.BlockSpec((BR, DIM), lambda j: (j, 0)),
            pl.BlockSpec((DIM, DIM), lambda j: (0, 0)),
            pl.BlockSpec((1, DIM), lambda j: (0, 0)),
            pl.BlockSpec((BR, 16), lambda j: (j, 0)),
        ],
        out_specs=pl.BlockSpec((N_GRAPHS, DIM), lambda j: (0, 0)),
        out_shape=jax.ShapeDtypeStruct((N_GRAPHS, DIM), jnp.float32),
        scratch_shapes=[
            pltpu.VMEM((N_GRAPHS, DIM), jnp.float32),
            pltpu.VMEM((N_GRAPHS, DIM), jnp.float32),
        ],
        compiler_params=pltpu.CompilerParams(
            dimension_semantics=("arbitrary",)),
    )(aggp, inorm, b, h_prev, wr, br, gid2)


def kernel(x, edge_index, graph_ids, W_init,
           W1, b1, Wr1, br1, W2, b2, Wr2, br2, W3, b3, Wr3, br3):
    src = edge_index[0]
    dst = edge_index[1]
    pad_e = jnp.full((EPAD - N_EDGES,), NPAD - 1, jnp.int32)
    src_pad = jnp.concatenate([src, pad_e])
    dst_pad = jnp.concatenate([dst, pad_e])
    src2d = src_pad.reshape(EB, 128)
    dst2d = dst_pad.reshape(EB, 128)
    src64 = src_pad.reshape(EB64, 64)
    dst64 = dst_pad.reshape(EB64, 64)
    x_pad = jnp.pad(x, ((0, NPAD - N_NODES), (0, 0)))
    gid_pad = jnp.concatenate(
        [graph_ids, jnp.full((NPAD - N_NODES,), -1, jnp.int32)])
    gid2 = jnp.broadcast_to(gid_pad[:, None], (NPAD, 16))
    ones128 = jnp.ones((128, 128), jnp.float32)
    zeros_nd = jnp.zeros((NPAD, DIM), jnp.float32)
    b1r = b1.reshape(1, DIM)
    br1r = br1.reshape(1, DIM)
    b2r = b2.reshape(1, DIM)
    br2r = br2.reshape(1, DIM)
    b3r = b3.reshape(1, DIM)
    br3r = br3.reshape(1, DIM)

    idx_all = jnp.stack([src2d, dst2d])
    degp = _sc_degrees(idx_all, ones128, zeros_nd)
    h0, m1, onorm, inorm = _tc_embed(degp, x_pad, W_init, W1)
    aggp = _sc_scatter(m1, src2d, dst2d, zeros_nd)
    h1, m2 = _tc_layer(aggp, inorm, b1r, h0, Wr1, br1r, onorm, W2)
    aggp = _sc_scatter(m2, src2d, dst2d, zeros_nd)
    h2, m3 = _tc_layer(aggp, inorm, b2r, h1, Wr2, br2r, onorm, W3)
    aggp = _sc_scatter(m3, src2d, dst2d, zeros_nd)
    return _tc_final(aggp, inorm, b3r, h2, Wr3, br3r, gid2)
